# baseline (speedup 1.0000x reference)
import jax
import jax.numpy as jnp
from jax import lax
from jax.experimental import pallas as pl
from jax.experimental.pallas import tpu as pltpu

_B, _P, _H, _D = 16, 16, 32, 128
_S = 4096
_SBLK = 512


def _body(pos_ref, sel_ref, k_ref, v_ref, ko_ref, vo_ref):
    base = pl.program_id(1) * _SBLK
    ko_ref[...] = jnp.zeros_like(ko_ref)
    vo_ref[...] = jnp.zeros_like(vo_ref)

    def body(p, c):
        dst = pos_ref[p] - base
        src = sel_ref[p]

        @pl.when(jnp.logical_and(dst >= 0, dst < _SBLK))
        def _():
            ko_ref[dst] = k_ref[src]
            vo_ref[dst] = v_ref[src]

        return c

    lax.fori_loop(0, _P, body, 0, unroll=True)


def kernel(k, v, pos, start_pos, max_pos, k_cache, v_cache):
    pos = pos.astype(jnp.int32)
    sel = (jnp.searchsorted(pos, pos, side="right") - 1).astype(jnp.int32)
    k2 = lax.bitcast_convert_type(k.reshape(_B * _P, 16, 128, 2), jnp.int32)
    v2 = lax.bitcast_convert_type(v.reshape(_B * _P, 16, 128, 2), jnp.int32)

    nsb = _S // _SBLK
    ko, vo = pl.pallas_call(
        _body,
        grid=(_B, nsb),
        in_specs=[
            pl.BlockSpec(memory_space=pltpu.SMEM),
            pl.BlockSpec(memory_space=pltpu.SMEM),
            pl.BlockSpec((_P, 16, 128), lambda b, s: (b, 0, 0)),
            pl.BlockSpec((_P, 16, 128), lambda b, s: (b, 0, 0)),
        ],
        out_specs=[
            pl.BlockSpec((_SBLK, 16, 128), lambda b, s: (b * nsb + s, 0, 0)),
            pl.BlockSpec((_SBLK, 16, 128), lambda b, s: (b * nsb + s, 0, 0)),
        ],
        out_shape=[jax.ShapeDtypeStruct((_B * _S, 16, 128), jnp.int32)] * 2,
        compiler_params=pltpu.CompilerParams(
            dimension_semantics=("parallel", "parallel"),
        ),
    )(pos, sel, k2, v2)
    ko = lax.bitcast_convert_type(ko, jnp.float16).reshape(_B, _S, _H, _D)
    vo = lax.bitcast_convert_type(vo, jnp.float16).reshape(_B, _S, _H, _D)
    return (ko, vo)


# speedup vs baseline: 8.0439x; 8.0439x over previous
import jax
import jax.numpy as jnp
from jax import lax
from jax.experimental import pallas as pl
from jax.experimental.pallas import tpu as pltpu

_B, _P, _H, _D = 16, 16, 32, 128
_S = 4096
_ZR = 512
_NSB = _S // _ZR


def _body(pos_ref, sel_ref, zref, k_ref, v_ref, ko_ref, vo_ref, sk, sv, sr):
    i = pl.program_id(0)
    b = i // _NSB
    s0 = (i % _NSB) * _ZR

    ck = pltpu.make_async_copy(zref, ko_ref.at[b, pl.ds(s0, _ZR)], sk)
    cv = pltpu.make_async_copy(zref, vo_ref.at[b, pl.ds(s0, _ZR)], sv)
    ck.start()
    cv.start()
    ck.wait()
    cv.wait()

    def body(p, c):
        dst = pos_ref[p] - s0
        src = sel_ref[p]

        @pl.when(jnp.logical_and(dst >= 0, dst < _ZR))
        def _():
            rk = pltpu.make_async_copy(
                k_ref.at[0, src], ko_ref.at[b, pos_ref[p]], sr)
            rv = pltpu.make_async_copy(
                v_ref.at[0, src], vo_ref.at[b, pos_ref[p]], sr)
            rk.start()
            rv.start()
            rk.wait()
            rv.wait()

        return c

    lax.fori_loop(0, _P, body, 0, unroll=True)


def kernel(k, v, pos, start_pos, max_pos, k_cache, v_cache):
    pos = pos.astype(jnp.int32)
    sel = (jnp.searchsorted(pos, pos, side="right") - 1).astype(jnp.int32)
    kb = lax.bitcast_convert_type(k, jnp.bfloat16)
    vb = lax.bitcast_convert_type(v, jnp.bfloat16)
    zeros = jnp.zeros((_ZR, _H, _D), dtype=jnp.bfloat16)

    ko, vo = pl.pallas_call(
        _body,
        grid=(_B * _NSB,),
        in_specs=[
            pl.BlockSpec(memory_space=pltpu.SMEM),
            pl.BlockSpec(memory_space=pltpu.SMEM),
            pl.BlockSpec((_ZR, _H, _D), lambda i: (0, 0, 0)),
            pl.BlockSpec((1, _P, _H, _D), lambda i: (i // _NSB, 0, 0, 0)),
            pl.BlockSpec((1, _P, _H, _D), lambda i: (i // _NSB, 0, 0, 0)),
        ],
        out_specs=[
            pl.BlockSpec(memory_space=pl.ANY),
            pl.BlockSpec(memory_space=pl.ANY),
        ],
        out_shape=[jax.ShapeDtypeStruct((_B, _S, _H, _D), jnp.bfloat16)] * 2,
        scratch_shapes=[
            pltpu.SemaphoreType.DMA,
            pltpu.SemaphoreType.DMA,
            pltpu.SemaphoreType.DMA,
        ],
        compiler_params=pltpu.CompilerParams(
            dimension_semantics=("arbitrary",),
        ),
    )(pos, sel, zeros, kb, vb)
    return (lax.bitcast_convert_type(ko, jnp.float16),
            lax.bitcast_convert_type(vo, jnp.float16))


# speedup vs baseline: 8.6230x; 1.0720x over previous
import jax
import jax.numpy as jnp
from jax import lax
from jax.experimental import pallas as pl
from jax.experimental.pallas import tpu as pltpu

_B, _P, _H, _D = 16, 16, 32, 128
_S = 4096
_ZR = 512
_NSB = _S // _ZR
_N = _B * _NSB
_DEPTH = 4


def _body(pos_ref, sel_ref, zref, k_ref, v_ref, ko_ref, vo_ref, sk, sv, sr):
    i = pl.program_id(0)

    @pl.when(i < _N)
    def _fill():
        b = i // _NSB
        s0 = (i % _NSB) * _ZR
        pltpu.make_async_copy(
            zref, ko_ref.at[b, pl.ds(s0, _ZR)], sk.at[i % _DEPTH]).start()
        pltpu.make_async_copy(
            zref, vo_ref.at[b, pl.ds(s0, _ZR)], sv.at[i % _DEPTH]).start()

    @pl.when(i >= _DEPTH)
    def _drain():
        j = i - _DEPTH
        b = j // _NSB
        s0 = (j % _NSB) * _ZR
        pltpu.make_async_copy(
            zref, ko_ref.at[b, pl.ds(s0, _ZR)], sk.at[j % _DEPTH]).wait()
        pltpu.make_async_copy(
            zref, vo_ref.at[b, pl.ds(s0, _ZR)], sv.at[j % _DEPTH]).wait()

        def body(p, c):
            dst = pos_ref[p] - s0
            src = sel_ref[p]

            @pl.when(jnp.logical_and(dst >= 0, dst < _ZR))
            def _():
                rk = pltpu.make_async_copy(
                    k_ref.at[0, src], ko_ref.at[b, pos_ref[p]], sr)
                rv = pltpu.make_async_copy(
                    v_ref.at[0, src], vo_ref.at[b, pos_ref[p]], sr)
                rk.start()
                rv.start()
                rk.wait()
                rv.wait()

            return c

        lax.fori_loop(0, _P, body, 0, unroll=True)


def kernel(k, v, pos, start_pos, max_pos, k_cache, v_cache):
    pos = pos.astype(jnp.int32)
    sel = (jnp.searchsorted(pos, pos, side="right") - 1).astype(jnp.int32)
    kb = lax.bitcast_convert_type(k, jnp.bfloat16)
    vb = lax.bitcast_convert_type(v, jnp.bfloat16)
    zeros = jnp.zeros((_ZR, _H, _D), dtype=jnp.bfloat16)

    ko, vo = pl.pallas_call(
        _body,
        grid=(_N + _DEPTH,),
        in_specs=[
            pl.BlockSpec(memory_space=pltpu.SMEM),
            pl.BlockSpec(memory_space=pltpu.SMEM),
            pl.BlockSpec((_ZR, _H, _D), lambda i: (0, 0, 0)),
            pl.BlockSpec((1, _P, _H, _D),
                         lambda i: (jnp.maximum(i - _DEPTH, 0) // _NSB, 0, 0, 0)),
            pl.BlockSpec((1, _P, _H, _D),
                         lambda i: (jnp.maximum(i - _DEPTH, 0) // _NSB, 0, 0, 0)),
        ],
        out_specs=[
            pl.BlockSpec(memory_space=pl.ANY),
            pl.BlockSpec(memory_space=pl.ANY),
        ],
        out_shape=[jax.ShapeDtypeStruct((_B, _S, _H, _D), jnp.bfloat16)] * 2,
        scratch_shapes=[
            pltpu.SemaphoreType.DMA((_DEPTH,)),
            pltpu.SemaphoreType.DMA((_DEPTH,)),
            pltpu.SemaphoreType.DMA,
        ],
        compiler_params=pltpu.CompilerParams(
            dimension_semantics=("arbitrary",),
        ),
    )(pos, sel, zeros, kb, vb)
    return (lax.bitcast_convert_type(ko, jnp.float16),
            lax.bitcast_convert_type(vo, jnp.float16))
